# baseline (speedup 1.0000x reference)
import functools

import jax
import jax.numpy as jnp
from jax import lax
from jax.experimental import pallas as pl
from jax.experimental.pallas import tpu as pltpu
from jax.experimental.pallas import tpu_sc as plsc

_GATH = 128
_CHUNK = 256
_NBUF = 3


@functools.lru_cache(maxsize=None)
def _make_gather(num_rows, dim, table_rows):
    info = plsc.get_sparse_core_info()
    nc, ns = info.num_cores, info.num_subcores
    nw = nc * ns
    assert num_rows % (nw * _CHUNK) == 0
    cpw = num_rows // (nw * _CHUNK)

    mesh = plsc.VectorSubcoreMesh(core_axis_name="c", subcore_axis_name="s")

    @functools.partial(
        pl.kernel,
        mesh=mesh,
        out_type=jax.ShapeDtypeStruct((num_rows, dim), jnp.float32),
        scratch_types=(
            [pltpu.VMEM((cpw * _CHUNK,), jnp.int32)]
            + [pltpu.VMEM((_CHUNK, dim), jnp.float32)] * _NBUF
            + [pltpu.SemaphoreType.DMA] * (2 * _NBUF)
        ),
        compiler_params=pltpu.CompilerParams(use_tc_tiling_on_sc=True),
    )
    def gather_k(table_hbm, idx_hbm, out_hbm, idx_v, *bufs_and_sems):
        bufs = bufs_and_sems[:_NBUF]
        sems = bufs_and_sems[_NBUF:2 * _NBUF]
        wsems = bufs_and_sems[2 * _NBUF:]
        wid = lax.axis_index("s") * nc + lax.axis_index("c")
        base = wid * cpw
        pltpu.sync_copy(idx_hbm.at[pl.ds(base * _CHUNK, cpw * _CHUNK)], idx_v)

        def gather_copies(t, b):
            return [
                pltpu.make_async_copy(
                    table_hbm.at[idx_v.at[pl.ds(t * _CHUNK + off, _GATH)]],
                    bufs[b].at[pl.ds(off, _GATH)], sems[b])
                for off in range(0, _CHUNK, _GATH)
            ]

        def start_chunk(t, b):
            for c in gather_copies(t, b):
                c.start()

        def write_copy(t, b):
            return pltpu.make_async_copy(
                bufs[b], out_hbm.at[pl.ds((base + t) * _CHUNK, _CHUNK)],
                wsems[b])

        assert cpw > _NBUF and (cpw - 1) % _NBUF == 0
        for p in range(_NBUF - 1):
            start_chunk(p, p)

        for c in gather_copies(0, 0):
            c.wait()
        write_copy(0, 0).start()
        start_chunk(_NBUF - 1, _NBUF - 1)

        def outer(tt, carry):
            for k in range(_NBUF):
                t = tt * _NBUF + k + 1
                b = (k + 1) % _NBUF
                bn = (b + _NBUF - 1) % _NBUF
                for c in gather_copies(t, b):
                    c.wait()
                write_copy(t, b).start()

                @pl.when(t + _NBUF - 1 < cpw)
                def _():
                    write_copy(t - 1, bn).wait()
                    start_chunk(t + _NBUF - 1, bn)
            return carry

        lax.fori_loop(0, (cpw - 1) // _NBUF, outer, 0)

        for t in range(cpw - _NBUF, cpw):
            write_copy(t, t % _NBUF).wait()

    return gather_k


def kernel(indices, weight):
    table_rows, dim = weight.shape
    batch, seq = indices.shape
    idx = indices.T.reshape(-1).astype(jnp.int32)
    out = _make_gather(batch * seq, dim, table_rows)(weight, idx)
    return out.reshape(seq, batch, dim).transpose(1, 0, 2)


# speedup vs baseline: 1.0104x; 1.0104x over previous
import functools

import jax
import jax.numpy as jnp
from jax import lax
from jax.experimental import pallas as pl
from jax.experimental.pallas import tpu as pltpu
from jax.experimental.pallas import tpu_sc as plsc

_GATH = 64
_CHUNK = 256
_NBUF = 3


@functools.lru_cache(maxsize=None)
def _make_gather(num_rows, dim, table_rows):
    info = plsc.get_sparse_core_info()
    nc, ns = info.num_cores, info.num_subcores
    nw = nc * ns
    assert num_rows % (nw * _CHUNK) == 0
    cpw = num_rows // (nw * _CHUNK)

    mesh = plsc.VectorSubcoreMesh(core_axis_name="c", subcore_axis_name="s")

    @functools.partial(
        pl.kernel,
        mesh=mesh,
        out_type=jax.ShapeDtypeStruct((num_rows, dim), jnp.float32),
        scratch_types=(
            [pltpu.VMEM((cpw * _CHUNK,), jnp.int32)]
            + [pltpu.VMEM((_CHUNK, dim), jnp.float32)] * _NBUF
            + [pltpu.SemaphoreType.DMA] * _NBUF
        ),
        compiler_params=pltpu.CompilerParams(use_tc_tiling_on_sc=True),
    )
    def gather_k(table_hbm, idx_hbm, out_hbm, idx_v, *bufs_and_sems):
        bufs = bufs_and_sems[:_NBUF]
        sems = bufs_and_sems[_NBUF:]
        wid = lax.axis_index("s") * nc + lax.axis_index("c")
        base = wid * cpw
        pltpu.sync_copy(idx_hbm.at[pl.ds(base * _CHUNK, cpw * _CHUNK)], idx_v)

        def gather_copies(t, b):
            return [
                pltpu.make_async_copy(
                    table_hbm.at[idx_v.at[pl.ds(t * _CHUNK + off, _GATH)]],
                    bufs[b].at[pl.ds(off, _GATH)], sems[b])
                for off in range(0, _CHUNK, _GATH)
            ]

        def start_chunk(t, b):
            for c in gather_copies(t, b):
                c.start()

        def finish_chunk(t, b):
            for c in gather_copies(t, b):
                c.wait()
            pltpu.sync_copy(
                bufs[b], out_hbm.at[pl.ds((base + t) * _CHUNK, _CHUNK)])

        for p in range(_NBUF - 1):
            start_chunk(p, p)

        def outer(tt, carry):
            for b in range(_NBUF):
                t = tt * _NBUF + b
                for c in gather_copies(t, b):
                    c.wait()

                @pl.when(t + _NBUF - 1 < cpw)
                def _():
                    start_chunk(t + _NBUF - 1, (b + _NBUF - 1) % _NBUF)

                pltpu.sync_copy(
                    bufs[b], out_hbm.at[pl.ds((base + t) * _CHUNK, _CHUNK)])
            return carry

        lax.fori_loop(0, cpw // _NBUF, outer, 0)

        for t in range(cpw - cpw % _NBUF, cpw):
            finish_chunk(t, t % _NBUF)

    return gather_k


def kernel(indices, weight):
    table_rows, dim = weight.shape
    batch, seq = indices.shape
    idx = indices.T.reshape(-1).astype(jnp.int32)
    out = _make_gather(batch * seq, dim, table_rows)(weight, idx)
    return out.reshape(seq, batch, dim).transpose(1, 0, 2)


# speedup vs baseline: 1.0128x; 1.0024x over previous
import functools

import jax
import jax.numpy as jnp
from jax import lax
from jax.experimental import pallas as pl
from jax.experimental.pallas import tpu as pltpu
from jax.experimental.pallas import tpu_sc as plsc

_GATH = 128
_CHUNK = 320
_NBUF = 3


@functools.lru_cache(maxsize=None)
def _make_gather(num_rows, dim, table_rows):
    info = plsc.get_sparse_core_info()
    nc, ns = info.num_cores, info.num_subcores
    nw = nc * ns
    assert num_rows % (nw * _CHUNK) == 0
    cpw = num_rows // (nw * _CHUNK)

    mesh = plsc.VectorSubcoreMesh(core_axis_name="c", subcore_axis_name="s")

    @functools.partial(
        pl.kernel,
        mesh=mesh,
        out_type=jax.ShapeDtypeStruct((num_rows, dim), jnp.float32),
        scratch_types=(
            [pltpu.VMEM((cpw * _CHUNK,), jnp.int32)]
            + [pltpu.VMEM((_CHUNK, dim), jnp.float32)] * _NBUF
            + [pltpu.SemaphoreType.DMA] * _NBUF
        ),
        compiler_params=pltpu.CompilerParams(use_tc_tiling_on_sc=True),
    )
    def gather_k(table_hbm, idx_hbm, out_hbm, idx_v, *bufs_and_sems):
        bufs = bufs_and_sems[:_NBUF]
        sems = bufs_and_sems[_NBUF:]
        wid = lax.axis_index("s") * nc + lax.axis_index("c")
        base = wid * cpw
        pltpu.sync_copy(idx_hbm.at[pl.ds(base * _CHUNK, cpw * _CHUNK)], idx_v)

        def gather_copies(t, b):
            return [
                pltpu.make_async_copy(
                    table_hbm.at[idx_v.at[
                        pl.ds(t * _CHUNK + off, min(_GATH, _CHUNK - off))]],
                    bufs[b].at[pl.ds(off, min(_GATH, _CHUNK - off))], sems[b])
                for off in range(0, _CHUNK, _GATH)
            ]

        def start_chunk(t, b):
            for c in gather_copies(t, b):
                c.start()

        def finish_chunk(t, b):
            for c in gather_copies(t, b):
                c.wait()
            pltpu.sync_copy(
                bufs[b], out_hbm.at[pl.ds((base + t) * _CHUNK, _CHUNK)])

        for p in range(_NBUF - 1):
            start_chunk(p, p)

        def outer(tt, carry):
            for b in range(_NBUF):
                t = tt * _NBUF + b
                for c in gather_copies(t, b):
                    c.wait()

                @pl.when(t + _NBUF - 1 < cpw)
                def _():
                    start_chunk(t + _NBUF - 1, (b + _NBUF - 1) % _NBUF)

                pltpu.sync_copy(
                    bufs[b], out_hbm.at[pl.ds((base + t) * _CHUNK, _CHUNK)])
            return carry

        lax.fori_loop(0, cpw // _NBUF, outer, 0)

        for t in range(cpw - cpw % _NBUF, cpw):
            finish_chunk(t, t % _NBUF)

    return gather_k


def kernel(indices, weight):
    table_rows, dim = weight.shape
    batch, seq = indices.shape
    idx = indices.T.reshape(-1).astype(jnp.int32)
    out = _make_gather(batch * seq, dim, table_rows)(weight, idx)
    return out.reshape(seq, batch, dim).transpose(1, 0, 2)
